# baseline (speedup 1.0000x reference)
import functools
import jax
import jax.numpy as jnp
from jax import lax
from jax.experimental import pallas as pl
from jax.experimental.pallas import tpu as pltpu, tpu_sc as plsc

N = 32768
D = 4096
K = 2048
BR = 1024
NW = 32
CHUNK = N // NW
CH = 8


def _matvec_body(x_ref, v_ref, y_ref):
    y_ref[...] = jnp.dot(x_ref[...], v_ref[...],
                         preferred_element_type=jnp.float32)


def _weights_body(d_ref, s_ref, w_ref):
    d = d_ref[...]
    bits = jax.lax.bitcast_convert_type(d, jnp.int32)
    sign = jnp.int32(-2147483648)
    key = bits ^ jax.lax.shift_right_logical(
        jax.lax.shift_right_arithmetic(bits, 31), 1)
    biased = jnp.int32(0)
    for b in range(31, -1, -1):
        cand_biased = biased | jnp.int32((1 << b) - (1 << 32 if b == 31 else 0))
        cand_int = cand_biased ^ sign
        cnt = jnp.sum((key >= cand_int).astype(jnp.int32))
        biased = jnp.where(cnt >= K, cand_biased, biased)
    thresh = biased ^ sign

    mask_gt = key > thresh
    tie = key == thresh
    need = K - jnp.sum(mask_gt.astype(jnp.int32))

    row = jax.lax.broadcasted_iota(jnp.int32, (256, 128), 0)
    col = jax.lax.broadcasted_iota(jnp.int32, (256, 128), 1)
    gidx = row * 128 + col
    tie_i = tie.astype(jnp.int32)
    p = jnp.int32(0)
    for b in range(14, -1, -1):
        j_cand = p + jnp.int32((1 << b) - 1)
        f = jnp.sum(jnp.where(gidx <= j_cand, tie_i, 0))
        p = jnp.where(f >= need, p, p + jnp.int32(1 << b))
    tie_sel = tie & (gidx <= p) & (need > 0)

    sel = mask_gt | tie_sel
    v = s_ref[...]
    y = d / jnp.sqrt(jnp.sum(v * v))
    w = 1.0 / (1.0 + jnp.exp(-y))
    w_ref[...] = jnp.where(sel, w, 0.0)


def _sc_gather_body(w_hbm, x_hbm, out_hbm, wchunk, idx_v, w_v, rows_v, rows_b,
                    acc, sem, sem_b):
    wid = lax.axis_index("s") * 2 + lax.axis_index("c")
    base = wid * CHUNK
    pltpu.sync_copy(w_hbm.at[pl.ds(base, CHUNK)], wchunk)

    zero16 = jnp.zeros((16,), jnp.float32)
    zero16i = jnp.zeros((16,), jnp.int32)

    def _zacc(s, carry):
        acc[pl.ds(s * 16, 16)] = zero16
        return carry

    lax.fori_loop(0, D // 16, _zacc, 0)

    def _compact(j, cnt):
        vec = wchunk[pl.ds(j * 16, 16)]
        for r in range(16):
            s_w = vec[r]
            idx_v[pl.ds(cnt, 16)] = zero16i + (base + j * 16 + r)
            w_v[pl.ds(cnt, 16)] = zero16 + s_w
            cnt = cnt + jnp.where(s_w > 0.0, 1, 0).astype(jnp.int32)
        return cnt

    ca = lax.fori_loop(0, CHUNK // 16, _compact, jnp.int32(0))
    idx_v[pl.ds(ca, 16)] = zero16i
    w_v[pl.ds(ca, 16)] = zero16

    def _fire(off, buf, s):
        pltpu.async_copy(x_hbm.at[idx_v.at[pl.ds(off, CH)]], buf, s)

    def _drain(buf, s):
        pltpu.make_async_copy(x_hbm.at[idx_v.at[pl.ds(0, CH)]], buf, s).wait()

    def _accum(off, buf):
        wvec = w_v[pl.ds(off, 16)]

        def _cols(sg, carry2):
            for u in range(2):
                o = sg * 32 + u * 16
                t = wvec[0] * buf[0, pl.ds(o, 16)]
                for r in range(1, CH):
                    t = t + wvec[r] * buf[r, pl.ds(o, 16)]
                plsc.addupdate(acc.at[pl.ds(o, 16)], t)
            return carry2

        lax.fori_loop(0, D // 32, _cols, 0)

    def _gather_segment(nb):

        @pl.when(nb > 0)
        def _():
            _fire(0, rows_v, sem)

        def _pair(h, carry):
            g0 = h * 2

            @pl.when(g0 + 1 < nb)
            def _():
                _fire((g0 + 1) * CH, rows_b, sem_b)

            _drain(rows_v, sem)
            _accum(g0 * CH, rows_v)

            @pl.when(g0 + 2 < nb)
            def _():
                _fire((g0 + 2) * CH, rows_v, sem)

            @pl.when(g0 + 1 < nb)
            def _():
                _drain(rows_b, sem_b)
                _accum((g0 + 1) * CH, rows_b)

            return carry

        lax.fori_loop(0, (nb + 1) // 2, _pair, 0)

    _gather_segment((ca + (CH - 1)) // CH)
    pltpu.sync_copy(acc, out_hbm.at[wid])


def _reduce_body(p_ref, o_ref):
    o_ref[...] = jnp.sum(p_ref[...], axis=0, keepdims=True)


def kernel(x, learnable_vector):
    nblk = N // BR
    d = pl.pallas_call(
        _matvec_body,
        grid=(nblk,),
        in_specs=[
            pl.BlockSpec((BR, D), lambda i: (i, 0)),
            pl.BlockSpec((D, 1), lambda i: (0, 0)),
        ],
        out_specs=pl.BlockSpec((BR, 1), lambda i: (i, 0)),
        out_shape=jax.ShapeDtypeStruct((N, 1), jnp.float32),
    )(x, learnable_vector)

    d2 = d.reshape(256, 128)
    w2 = pl.pallas_call(
        _weights_body,
        in_specs=[
            pl.BlockSpec((256, 128), lambda: (0, 0)),
            pl.BlockSpec((32, 128), lambda: (0, 0)),
        ],
        out_specs=pl.BlockSpec((256, 128), lambda: (0, 0)),
        out_shape=jax.ShapeDtypeStruct((256, 128), jnp.float32),
    )(d2, learnable_vector.reshape(32, 128))
    sc_gather = pl.kernel(
        _sc_gather_body,
        out_type=jax.ShapeDtypeStruct((NW, D), jnp.float32),
        mesh=plsc.VectorSubcoreMesh(core_axis_name="c", subcore_axis_name="s"),
        scratch_types=[
            pltpu.VMEM((CHUNK,), jnp.float32),
            pltpu.VMEM((CHUNK + 16,), jnp.int32),
            pltpu.VMEM((CHUNK + 16,), jnp.float32),
            pltpu.VMEM((CH, D), jnp.float32),
            pltpu.VMEM((CH, D), jnp.float32),
            pltpu.VMEM((D,), jnp.float32),
            pltpu.SemaphoreType.DMA,
            pltpu.SemaphoreType.DMA,
        ],
    )
    parts = sc_gather(w2.reshape(N), x)

    out = pl.pallas_call(
        _reduce_body,
        in_specs=[pl.BlockSpec((NW, D), lambda: (0, 0))],
        out_specs=pl.BlockSpec((1, D), lambda: (0, 0)),
        out_shape=jax.ShapeDtypeStruct((1, D), jnp.float32),
    )(parts)
    return out


# speedup vs baseline: 1.0178x; 1.0178x over previous
import jax
import jax.numpy as jnp
from jax import lax
from jax.experimental import pallas as pl
from jax.experimental.pallas import tpu as pltpu, tpu_sc as plsc

N = 32768
D = 4096
K = 2048
BR = 1024
NW = 32
CHUNK = N // NW
CH = 8


def _matvec_body(x_ref, v_ref, y_ref):
    y_ref[...] = jnp.dot(x_ref[...], v_ref[...],
                         preferred_element_type=jnp.float32)


def _weights_body(d_ref, s_ref, w_ref):
    d = d_ref[...]
    bits = jax.lax.bitcast_convert_type(d, jnp.int32)
    sign = jnp.int32(-2147483648)
    key = bits ^ jax.lax.shift_right_logical(
        jax.lax.shift_right_arithmetic(bits, 31), 1)
    biased = jnp.int32(0)
    for b in range(31, -1, -1):
        cand_biased = biased | jnp.int32((1 << b) - (1 << 32 if b == 31 else 0))
        cand_int = cand_biased ^ sign
        cnt = jnp.sum((key >= cand_int).astype(jnp.int32))
        biased = jnp.where(cnt >= K, cand_biased, biased)
    thresh = biased ^ sign

    mask_gt = key > thresh
    tie = key == thresh
    need = K - jnp.sum(mask_gt.astype(jnp.int32))

    row = jax.lax.broadcasted_iota(jnp.int32, (256, 128), 0)
    col = jax.lax.broadcasted_iota(jnp.int32, (256, 128), 1)
    gidx = row * 128 + col
    tie_i = tie.astype(jnp.int32)
    p = jnp.int32(0)
    for b in range(14, -1, -1):
        j_cand = p + jnp.int32((1 << b) - 1)
        f = jnp.sum(jnp.where(gidx <= j_cand, tie_i, 0))
        p = jnp.where(f >= need, p, p + jnp.int32(1 << b))
    tie_sel = tie & (gidx <= p) & (need > 0)

    sel = mask_gt | tie_sel
    v = s_ref[...]
    y = d / jnp.sqrt(jnp.sum(v * v))
    w = 1.0 / (1.0 + jnp.exp(-y))
    w_ref[...] = jnp.where(sel, w, 0.0)


def _sc_gather_body(w_hbm, x_hbm, out_hbm, wchunk, idx_v, w_v, rows_v, rows_b,
                    acc, sem, sem_b):
    wid = lax.axis_index("s") * 2 + lax.axis_index("c")
    base = wid * CHUNK
    pltpu.sync_copy(w_hbm.at[pl.ds(base, CHUNK)], wchunk)

    zero16 = jnp.zeros((16,), jnp.float32)
    zero16i = jnp.zeros((16,), jnp.int32)

    def _zacc(s, carry):
        acc[pl.ds(s * 16, 16)] = zero16
        return carry

    lax.fori_loop(0, D // 16, _zacc, 0)

    def _compact(j, cnt):
        vec = wchunk[pl.ds(j * 16, 16)]
        for r in range(16):
            s_w = vec[r]
            idx_v[pl.ds(cnt, 16)] = zero16i + (base + j * 16 + r)
            w_v[pl.ds(cnt, 16)] = zero16 + s_w
            cnt = cnt + jnp.where(s_w > 0.0, 1, 0).astype(jnp.int32)
        return cnt

    ca = lax.fori_loop(0, CHUNK // 16, _compact, jnp.int32(0))
    idx_v[pl.ds(ca, 16)] = zero16i
    w_v[pl.ds(ca, 16)] = zero16

    def _fire(off, buf, s):
        pltpu.async_copy(x_hbm.at[idx_v.at[pl.ds(off, CH)]], buf, s)

    def _drain(buf, s):
        pltpu.make_async_copy(x_hbm.at[idx_v.at[pl.ds(0, CH)]], buf, s).wait()

    def _accum(off, buf):
        wvec = w_v[pl.ds(off, 16)]

        def _cols(sg, carry2):
            t = wvec[0] * buf[0, pl.ds(sg * 16, 16)]
            for r in range(1, CH):
                t = t + wvec[r] * buf[r, pl.ds(sg * 16, 16)]
            plsc.addupdate(acc.at[pl.ds(sg * 16, 16)], t)
            return carry2

        lax.fori_loop(0, D // 16, _cols, 0)

    def _gather_segment(nb):

        @pl.when(nb > 0)
        def _():
            _fire(0, rows_v, sem)

        def _pair(h, carry):
            g0 = h * 2

            @pl.when(g0 + 1 < nb)
            def _():
                _fire((g0 + 1) * CH, rows_b, sem_b)

            _drain(rows_v, sem)
            _accum(g0 * CH, rows_v)

            @pl.when(g0 + 2 < nb)
            def _():
                _fire((g0 + 2) * CH, rows_v, sem)

            @pl.when(g0 + 1 < nb)
            def _():
                _drain(rows_b, sem_b)
                _accum((g0 + 1) * CH, rows_b)

            return carry

        lax.fori_loop(0, (nb + 1) // 2, _pair, 0)

    _gather_segment((ca + (CH - 1)) // CH)
    pltpu.sync_copy(acc, out_hbm.at[wid])


def _reduce_body(p_ref, o_ref):
    o_ref[...] = jnp.sum(p_ref[...], axis=0, keepdims=True)


def kernel(x, learnable_vector):
    nblk = N // BR
    d = pl.pallas_call(
        _matvec_body,
        grid=(nblk,),
        in_specs=[
            pl.BlockSpec((BR, D), lambda i: (i, 0)),
            pl.BlockSpec((D, 1), lambda i: (0, 0)),
        ],
        out_specs=pl.BlockSpec((BR, 1), lambda i: (i, 0)),
        out_shape=jax.ShapeDtypeStruct((N, 1), jnp.float32),
    )(x, learnable_vector)

    d2 = d.reshape(256, 128)
    w2 = pl.pallas_call(
        _weights_body,
        in_specs=[
            pl.BlockSpec((256, 128), lambda: (0, 0)),
            pl.BlockSpec((32, 128), lambda: (0, 0)),
        ],
        out_specs=pl.BlockSpec((256, 128), lambda: (0, 0)),
        out_shape=jax.ShapeDtypeStruct((256, 128), jnp.float32),
    )(d2, learnable_vector.reshape(32, 128))
    sc_gather = pl.kernel(
        _sc_gather_body,
        out_type=jax.ShapeDtypeStruct((NW, D), jnp.float32),
        mesh=plsc.VectorSubcoreMesh(core_axis_name="c", subcore_axis_name="s"),
        scratch_types=[
            pltpu.VMEM((CHUNK,), jnp.float32),
            pltpu.VMEM((CHUNK + 16,), jnp.int32),
            pltpu.VMEM((CHUNK + 16,), jnp.float32),
            pltpu.VMEM((CH, D), jnp.float32),
            pltpu.VMEM((CH, D), jnp.float32),
            pltpu.VMEM((D,), jnp.float32),
            pltpu.SemaphoreType.DMA,
            pltpu.SemaphoreType.DMA,
        ],
    )
    parts = sc_gather(w2.reshape(N), x)

    out = pl.pallas_call(
        _reduce_body,
        in_specs=[pl.BlockSpec((NW, D), lambda: (0, 0))],
        out_specs=pl.BlockSpec((1, D), lambda: (0, 0)),
        out_shape=jax.ShapeDtypeStruct((1, D), jnp.float32),
    )(parts)
    return out
